# Initial kernel scaffold; baseline (speedup 1.0000x reference)
import jax
import jax.numpy as jnp
from jax.experimental import pallas as pl


def kernel(x, table, offset):
    raise NotImplementedError("write your pallas kernel here")


# speedup vs baseline: 1.4885x; 1.4885x over previous
import jax
import jax.numpy as jnp
from jax.experimental import pallas as pl
from jax.experimental.pallas import tpu as pltpu

_BS = 512


def _body(off_ref, x_ref, t_ref, o_ref):
    del off_ref
    o_ref[...] = x_ref[...] + t_ref[...]


def kernel(x, table, offset=0):
    B, S, D = x.shape
    off = jnp.asarray(offset, jnp.int32).reshape((1,))
    grid = (S // _BS, B)
    spec = pltpu.PrefetchScalarGridSpec(
        num_scalar_prefetch=1,
        grid=grid,
        in_specs=[
            pl.BlockSpec((1, _BS, D), lambda i, j, off: (j, i, 0)),
            pl.BlockSpec((_BS, D), lambda i, j, off: (i + off[0] // _BS, 0)),
        ],
        out_specs=pl.BlockSpec((1, _BS, D), lambda i, j, off: (j, i, 0)),
    )
    return pl.pallas_call(
        _body,
        grid_spec=spec,
        out_shape=jax.ShapeDtypeStruct(x.shape, x.dtype),
        compiler_params=pltpu.CompilerParams(
            dimension_semantics=("arbitrary", "arbitrary"),
        ),
    )(off, x, table)


# speedup vs baseline: 1.6583x; 1.1141x over previous
import jax
import jax.numpy as jnp
from jax.experimental import pallas as pl
from jax.experimental.pallas import tpu as pltpu

_BS = 1024


def _body(off_ref, x_ref, t_ref, o_ref):
    del off_ref
    o_ref[...] = x_ref[...] + t_ref[...]


def kernel(x, table, offset=0):
    B, S, D = x.shape
    off = jnp.asarray(offset, jnp.int32).reshape((1,))
    grid = (S // _BS, B)
    spec = pltpu.PrefetchScalarGridSpec(
        num_scalar_prefetch=1,
        grid=grid,
        in_specs=[
            pl.BlockSpec((1, _BS, D), lambda i, j, off: (j, i, 0)),
            pl.BlockSpec((_BS, D), lambda i, j, off: (i + off[0] // _BS, 0)),
        ],
        out_specs=pl.BlockSpec((1, _BS, D), lambda i, j, off: (j, i, 0)),
    )
    return pl.pallas_call(
        _body,
        grid_spec=spec,
        out_shape=jax.ShapeDtypeStruct(x.shape, x.dtype),
        compiler_params=pltpu.CompilerParams(
            dimension_semantics=("arbitrary", "arbitrary"),
        ),
    )(off, x, table)


# speedup vs baseline: 1.7275x; 1.0417x over previous
import jax
import jax.numpy as jnp
from jax.experimental import pallas as pl
from jax.experimental.pallas import tpu as pltpu

_BS = 2048


def _body(off_ref, x_ref, t_ref, o_ref):
    del off_ref
    o_ref[...] = x_ref[...] + t_ref[...]


def kernel(x, table, offset=0):
    B, S, D = x.shape
    off = jnp.asarray(offset, jnp.int32).reshape((1,))
    grid = (S // _BS, B)
    spec = pltpu.PrefetchScalarGridSpec(
        num_scalar_prefetch=1,
        grid=grid,
        in_specs=[
            pl.BlockSpec((1, _BS, D), lambda i, j, off: (j, i, 0)),
            pl.BlockSpec((_BS, D), lambda i, j, off: (i + off[0] // _BS, 0)),
        ],
        out_specs=pl.BlockSpec((1, _BS, D), lambda i, j, off: (j, i, 0)),
    )
    return pl.pallas_call(
        _body,
        grid_spec=spec,
        out_shape=jax.ShapeDtypeStruct(x.shape, x.dtype),
        compiler_params=pltpu.CompilerParams(
            dimension_semantics=("arbitrary", "arbitrary"),
        ),
    )(off, x, table)
